# baseline (speedup 1.0000x reference)
import functools

import jax
import jax.numpy as jnp
from jax import lax
from jax.experimental import pallas as pl
from jax.experimental.pallas import tpu as pltpu
from jax.experimental.pallas import tpu_sc as plsc

N = 100000
NP = 100096
M = 16
W_LITS = 4
CHUNK = 512
FULL_CHUNKS = N // CHUNK
TAIL_N0 = FULL_CHUNKS * CHUNK
TAILP = NP - TAIL_N0
GROUP_UNROLL = 4


def _clause_rows_sc(a0, x1, x2, x1t, x2t):
  mesh = plsc.VectorSubcoreMesh(
      core_axis_name="c", subcore_axis_name="s", num_cores=2, num_subcores=16)

  @functools.partial(
      pl.kernel,
      mesh=mesh,
      compiler_params=pltpu.CompilerParams(needs_layout_passes=False),
      out_type=(
          jax.ShapeDtypeStruct((M, NP), jnp.float32),
          jax.ShapeDtypeStruct((M, NP), jnp.float32),
      ),
      scratch_types=[
          pltpu.VMEM((N,), jnp.float32),
          pltpu.VMEM((8, CHUNK), jnp.int32),
          pltpu.VMEM((8, CHUNK), jnp.int32),
          pltpu.VMEM((CHUNK,), jnp.float32),
          pltpu.VMEM((CHUNK,), jnp.float32),
          pltpu.SemaphoreType.DMA,
          pltpu.SemaphoreType.DMA,
          pltpu.SemaphoreType.DMA,
          pltpu.SemaphoreType.DMA,
      ],
  )
  def body(a_hbm, x1_hbm, x2_hbm, x1t_hbm, x2t_hbm, f1_hbm, f2_hbm,
           a_v, x_v0, x_v1, f_v0, f_v1, si0, si1, so0, so1):
    wid = lax.axis_index("s") * 2 + lax.axis_index("c")
    pltpu.sync_copy(a_hbm, a_v)
    x_bufs = (x_v0, x_v1)
    f_bufs = (f_v0, f_v1)
    si = (si0, si1)
    so = (so0, so1)

    def process(row, x_hbm, xt_hbm, f_hbm):
      def start_in(c, b):
        for k in range(W_LITS):
          pltpu.async_copy(x_hbm.at[row, k, pl.ds(0, 2), pl.ds(c * CHUNK, CHUNK)],
                           x_bufs[b].at[pl.ds(2 * k, 2)], si[b])

      def wait_in(b):
        for k in range(W_LITS):
          pltpu.make_async_copy(x_hbm.at[row, 0, pl.ds(0, 2), pl.ds(0, CHUNK)],
                                x_bufs[b].at[pl.ds(2 * k, 2)], si[b]).wait()

      def wait_out(b):
        pltpu.make_async_copy(f_bufs[b], f_hbm.at[row, pl.ds(0, CHUNK)],
                              so[b]).wait()

      def compute_group(x_v, f_v, g):
        fmax = None
        for w in range(W_LITS):
          i1 = x_v[2 * w, pl.ds(g * 16, 16)]
          i2 = x_v[2 * w + 1, pl.ds(g * 16, 16)]
          y1 = plsc.load_gather(a_v, [i1])
          y2 = plsc.load_gather(a_v, [i2])
          z = y1 * y2
          fmax = z if fmax is None else jnp.maximum(fmax, z)
        f_v[pl.ds(g * 16, 16)] = fmax

      def compute_chunk(b):
        x_v, f_v = x_bufs[b], f_bufs[b]
        n_groups = CHUNK // 16

        def grp_body(i, carry):
          for u in range(GROUP_UNROLL):
            compute_group(x_v, f_v, i * GROUP_UNROLL + u)
          return carry

        lax.fori_loop(0, n_groups // GROUP_UNROLL, grp_body, 0)

      def chunk_step(c, b):
        @pl.when(c + 1 < FULL_CHUNKS)
        def _():
          start_in(c + 1, 1 - b)

        wait_in(b)

        @pl.when(c >= 2)
        def _():
          wait_out(b)

        compute_chunk(b)
        pltpu.async_copy(f_bufs[b], f_hbm.at[row, pl.ds(c * CHUNK, CHUNK)],
                         so[b])

      start_in(0, 0)

      def loop_body(i, carry):
        chunk_step(2 * i, 0)
        chunk_step(2 * i + 1, 1)
        return carry

      lax.fori_loop(0, (FULL_CHUNKS - 1) // 2, loop_body, 0)
      chunk_step(FULL_CHUNKS - 1, 0)
      wait_out(1)
      wait_out(0)

      for k in range(8):
        pltpu.sync_copy(xt_hbm.at[row, k // 2, k % 2],
                        x_bufs[1].at[k, pl.ds(0, TAILP)])
      for g in range(TAILP // 16):
        compute_group(x_bufs[1], f_bufs[1], g)
      pltpu.sync_copy(f_bufs[1].at[pl.ds(0, TAILP)],
                      f_hbm.at[row, pl.ds(TAIL_N0, TAILP)])

    @pl.when(wid < M)
    def _():
      process(wid, x1_hbm, x1t_hbm, f1_hbm)

    @pl.when(wid >= M)
    def _():
      process(wid - M, x2_hbm, x2t_hbm, f2_hbm)

  return body(a0, x1, x2, x1t, x2t)


def _combine_tc(a0, w, f1, f2):
  def body(a_ref, w_ref, f1_ref, f2_ref, o_ref):
    wf = w_ref[...]
    wf = wf - jnp.max(wf)
    e = jnp.exp(wf)
    pi = e / jnp.sum(e)
    pi1 = jnp.sum(pi, axis=1).reshape(M, 1)
    pi2 = jnp.sum(pi, axis=0).reshape(M, 1)
    f1b = f1_ref[...]
    f2b = f2_ref[...]
    eu = jnp.sum(pi1 * f1b, axis=0, keepdims=True)
    ev = jnp.sum(pi2 * f2b, axis=0, keepdims=True)
    mm = jnp.dot(pi, f2b, preferred_element_type=jnp.float32)
    euv = jnp.sum(f1b * mm, axis=0, keepdims=True)
    fp = eu + ev - euv
    av = a_ref[...]
    o_ref[...] = av + fp - av * fp

  a_pad = jnp.pad(a0, (0, NP - N)).reshape(1, NP)
  out = pl.pallas_call(
      body,
      out_shape=jax.ShapeDtypeStruct((1, NP), jnp.float32),
  )(a_pad, w, f1, f2)
  return out.reshape(NP)[:N]


def kernel(a0, W, X1, X2):
  x1 = jnp.transpose(X1, (0, 2, 3, 1))
  x2 = jnp.transpose(X2, (0, 2, 3, 1))
  pad = ((0, 0), (0, 0), (0, 0), (0, TAILP - (N - TAIL_N0)))
  x1t = jnp.pad(x1[:, :, :, TAIL_N0:], pad)
  x2t = jnp.pad(x2[:, :, :, TAIL_N0:], pad)
  f1, f2 = _clause_rows_sc(a0, x1, x2, x1t, x2t)
  return _combine_tc(a0, W, f1, f2)


# speedup vs baseline: 1.1534x; 1.1534x over previous
import functools

import jax
import jax.numpy as jnp
from jax import lax
from jax.experimental import pallas as pl
from jax.experimental.pallas import tpu as pltpu
from jax.experimental.pallas import tpu_sc as plsc

N = 100000
NP = 100096
M = 16
W_LITS = 4
CHUNK = 1536
FULL_CHUNKS = N // CHUNK
TAIL_N0 = FULL_CHUNKS * CHUNK
TAILP = NP - TAIL_N0
GROUP_UNROLL = 8


def _clause_rows_sc(a0, x1, x2, x1t, x2t):
  mesh = plsc.VectorSubcoreMesh(
      core_axis_name="c", subcore_axis_name="s", num_cores=2, num_subcores=16)

  @functools.partial(
      pl.kernel,
      mesh=mesh,
      compiler_params=pltpu.CompilerParams(needs_layout_passes=False),
      out_type=(
          jax.ShapeDtypeStruct((M, NP), jnp.float32),
          jax.ShapeDtypeStruct((M, NP), jnp.float32),
      ),
      scratch_types=[
          pltpu.VMEM((N,), jnp.float32),
          pltpu.VMEM((8, CHUNK), jnp.int32),
          pltpu.VMEM((8, CHUNK), jnp.int32),
          pltpu.VMEM((CHUNK,), jnp.float32),
          pltpu.VMEM((CHUNK,), jnp.float32),
          pltpu.SemaphoreType.DMA,
          pltpu.SemaphoreType.DMA,
          pltpu.SemaphoreType.DMA,
          pltpu.SemaphoreType.DMA,
      ],
  )
  def body(a_hbm, x1_hbm, x2_hbm, x1t_hbm, x2t_hbm, f1_hbm, f2_hbm,
           a_v, x_v0, x_v1, f_v0, f_v1, si0, si1, so0, so1):
    wid = lax.axis_index("s") * 2 + lax.axis_index("c")
    pltpu.sync_copy(a_hbm, a_v)
    x_bufs = (x_v0, x_v1)
    f_bufs = (f_v0, f_v1)
    si = (si0, si1)
    so = (so0, so1)

    def process(row, x_hbm, xt_hbm, f_hbm):
      def start_in(c, b):
        for k in range(W_LITS):
          pltpu.async_copy(x_hbm.at[row, k, pl.ds(0, 2), pl.ds(c * CHUNK, CHUNK)],
                           x_bufs[b].at[pl.ds(2 * k, 2)], si[b])

      def wait_in(b):
        for k in range(W_LITS):
          pltpu.make_async_copy(x_hbm.at[row, 0, pl.ds(0, 2), pl.ds(0, CHUNK)],
                                x_bufs[b].at[pl.ds(2 * k, 2)], si[b]).wait()

      def wait_out(b):
        pltpu.make_async_copy(f_bufs[b], f_hbm.at[row, pl.ds(0, CHUNK)],
                              so[b]).wait()

      def compute_group(x_v, f_v, g):
        fmax = None
        for w in range(W_LITS):
          i1 = x_v[2 * w, pl.ds(g * 16, 16)]
          i2 = x_v[2 * w + 1, pl.ds(g * 16, 16)]
          y1 = plsc.load_gather(a_v, [i1])
          y2 = plsc.load_gather(a_v, [i2])
          z = y1 * y2
          fmax = z if fmax is None else jnp.maximum(fmax, z)
        f_v[pl.ds(g * 16, 16)] = fmax

      def compute_chunk(b):
        x_v, f_v = x_bufs[b], f_bufs[b]
        n_groups = CHUNK // 16

        def grp_body(i, carry):
          for u in range(GROUP_UNROLL):
            compute_group(x_v, f_v, i * GROUP_UNROLL + u)
          return carry

        lax.fori_loop(0, n_groups // GROUP_UNROLL, grp_body, 0)

      def chunk_step(c, b):
        @pl.when(c + 1 < FULL_CHUNKS)
        def _():
          start_in(c + 1, 1 - b)

        wait_in(b)

        @pl.when(c >= 2)
        def _():
          wait_out(b)

        compute_chunk(b)
        pltpu.async_copy(f_bufs[b], f_hbm.at[row, pl.ds(c * CHUNK, CHUNK)],
                         so[b])

      start_in(0, 0)

      def loop_body(i, carry):
        chunk_step(2 * i, 0)
        chunk_step(2 * i + 1, 1)
        return carry

      lax.fori_loop(0, (FULL_CHUNKS - 1) // 2, loop_body, 0)
      chunk_step(FULL_CHUNKS - 1, 0)
      wait_out(1)
      wait_out(0)

      for k in range(8):
        pltpu.sync_copy(xt_hbm.at[row, k // 2, k % 2],
                        x_bufs[1].at[k, pl.ds(0, TAILP)])
      for g in range(TAILP // 16):
        compute_group(x_bufs[1], f_bufs[1], g)
      pltpu.sync_copy(f_bufs[1].at[pl.ds(0, TAILP)],
                      f_hbm.at[row, pl.ds(TAIL_N0, TAILP)])

    @pl.when(wid < M)
    def _():
      process(wid, x1_hbm, x1t_hbm, f1_hbm)

    @pl.when(wid >= M)
    def _():
      process(wid - M, x2_hbm, x2t_hbm, f2_hbm)

  return body(a0, x1, x2, x1t, x2t)


def _combine_tc(a0, w, f1, f2):
  def body(a_ref, w_ref, f1_ref, f2_ref, o_ref):
    wf = w_ref[...]
    wf = wf - jnp.max(wf)
    e = jnp.exp(wf)
    pi = e / jnp.sum(e)
    pi1 = jnp.sum(pi, axis=1).reshape(M, 1)
    pi2 = jnp.sum(pi, axis=0).reshape(M, 1)
    f1b = f1_ref[...]
    f2b = f2_ref[...]
    eu = jnp.sum(pi1 * f1b, axis=0, keepdims=True)
    ev = jnp.sum(pi2 * f2b, axis=0, keepdims=True)
    mm = jnp.dot(pi, f2b, preferred_element_type=jnp.float32)
    euv = jnp.sum(f1b * mm, axis=0, keepdims=True)
    fp = eu + ev - euv
    av = a_ref[...]
    o_ref[...] = av + fp - av * fp

  a_pad = jnp.pad(a0, (0, NP - N)).reshape(1, NP)
  out = pl.pallas_call(
      body,
      out_shape=jax.ShapeDtypeStruct((1, NP), jnp.float32),
  )(a_pad, w, f1, f2)
  return out.reshape(NP)[:N]


def kernel(a0, W, X1, X2):
  x1 = jnp.transpose(X1, (0, 2, 3, 1))
  x2 = jnp.transpose(X2, (0, 2, 3, 1))
  pad = ((0, 0), (0, 0), (0, 0), (0, TAILP - (N - TAIL_N0)))
  x1t = jnp.pad(x1[:, :, :, TAIL_N0:], pad)
  x2t = jnp.pad(x2[:, :, :, TAIL_N0:], pad)
  f1, f2 = _clause_rows_sc(a0, x1, x2, x1t, x2t)
  return _combine_tc(a0, W, f1, f2)


# speedup vs baseline: 1.1619x; 1.0074x over previous
import functools

import jax
import jax.numpy as jnp
from jax import lax
from jax.experimental import pallas as pl
from jax.experimental.pallas import tpu as pltpu
from jax.experimental.pallas import tpu_sc as plsc

N = 100000
NP = 100096
M = 16
W_LITS = 4
CHUNK = 1536
FULL_CHUNKS = N // CHUNK
TAIL_N0 = FULL_CHUNKS * CHUNK
TAILP = NP - TAIL_N0
GROUP_UNROLL = 8


def _clause_rows_sc(a0, x1, x2, x1t, x2t):
  mesh = plsc.VectorSubcoreMesh(
      core_axis_name="c", subcore_axis_name="s", num_cores=2, num_subcores=16)

  @functools.partial(
      pl.kernel,
      mesh=mesh,
      compiler_params=pltpu.CompilerParams(needs_layout_passes=False),
      out_type=(
          jax.ShapeDtypeStruct((M, NP), jnp.float32),
          jax.ShapeDtypeStruct((M, NP), jnp.float32),
      ),
      scratch_types=[
          pltpu.VMEM((N,), jnp.float32),
          pltpu.VMEM((8, CHUNK), jnp.int32),
          pltpu.VMEM((8, CHUNK), jnp.int32),
          pltpu.VMEM((CHUNK,), jnp.float32),
          pltpu.VMEM((CHUNK,), jnp.float32),
          pltpu.SemaphoreType.DMA,
          pltpu.SemaphoreType.DMA,
          pltpu.SemaphoreType.DMA,
          pltpu.SemaphoreType.DMA,
      ],
  )
  def body(a_hbm, x1_hbm, x2_hbm, x1t_hbm, x2t_hbm, f1_hbm, f2_hbm,
           a_v, x_v0, x_v1, f_v0, f_v1, si0, si1, so0, so1):
    wid = lax.axis_index("s") * 2 + lax.axis_index("c")
    a_copy = pltpu.async_copy(a_hbm, a_v, so0)
    x_bufs = (x_v0, x_v1)
    f_bufs = (f_v0, f_v1)
    si = (si0, si1)
    so = (so0, so1)

    def process(row, x_hbm, xt_hbm, f_hbm):
      def start_in(c, b):
        for k in range(W_LITS):
          pltpu.async_copy(x_hbm.at[row, k, pl.ds(0, 2), pl.ds(c * CHUNK, CHUNK)],
                           x_bufs[b].at[pl.ds(2 * k, 2)], si[b])

      def wait_in(b):
        for k in range(W_LITS):
          pltpu.make_async_copy(x_hbm.at[row, 0, pl.ds(0, 2), pl.ds(0, CHUNK)],
                                x_bufs[b].at[pl.ds(2 * k, 2)], si[b]).wait()

      def wait_out(b):
        pltpu.make_async_copy(f_bufs[b], f_hbm.at[row, pl.ds(0, CHUNK)],
                              so[b]).wait()

      def compute_group(x_v, f_v, g):
        fmax = None
        for w in range(W_LITS):
          i1 = x_v[2 * w, pl.ds(g * 16, 16)]
          i2 = x_v[2 * w + 1, pl.ds(g * 16, 16)]
          y1 = plsc.load_gather(a_v, [i1])
          y2 = plsc.load_gather(a_v, [i2])
          z = y1 * y2
          fmax = z if fmax is None else jnp.maximum(fmax, z)
        f_v[pl.ds(g * 16, 16)] = fmax

      def compute_chunk(b):
        x_v, f_v = x_bufs[b], f_bufs[b]
        n_groups = CHUNK // 16

        def grp_body(i, carry):
          for u in range(GROUP_UNROLL):
            compute_group(x_v, f_v, i * GROUP_UNROLL + u)
          return carry

        lax.fori_loop(0, n_groups // GROUP_UNROLL, grp_body, 0)

      def chunk_step(c, b):
        @pl.when(c + 1 < FULL_CHUNKS)
        def _():
          start_in(c + 1, 1 - b)

        wait_in(b)

        @pl.when(c >= 2)
        def _():
          wait_out(b)

        compute_chunk(b)
        pltpu.async_copy(f_bufs[b], f_hbm.at[row, pl.ds(c * CHUNK, CHUNK)],
                         so[b])

      start_in(0, 0)
      a_copy.wait()

      def loop_body(i, carry):
        chunk_step(2 * i, 0)
        chunk_step(2 * i + 1, 1)
        return carry

      lax.fori_loop(0, (FULL_CHUNKS - 1) // 2, loop_body, 0)
      chunk_step(FULL_CHUNKS - 1, 0)
      wait_out(1)
      wait_out(0)

      for k in range(8):
        pltpu.sync_copy(xt_hbm.at[row, k // 2, k % 2],
                        x_bufs[1].at[k, pl.ds(0, TAILP)])
      for g in range(TAILP // 16):
        compute_group(x_bufs[1], f_bufs[1], g)
      pltpu.sync_copy(f_bufs[1].at[pl.ds(0, TAILP)],
                      f_hbm.at[row, pl.ds(TAIL_N0, TAILP)])

    @pl.when(wid < M)
    def _():
      process(wid, x1_hbm, x1t_hbm, f1_hbm)

    @pl.when(wid >= M)
    def _():
      process(wid - M, x2_hbm, x2t_hbm, f2_hbm)

  return body(a0, x1, x2, x1t, x2t)


def _combine_tc(a0, w, f1, f2):
  def body(a_ref, w_ref, f1_ref, f2_ref, o_ref):
    wf = w_ref[...]
    wf = wf - jnp.max(wf)
    e = jnp.exp(wf)
    pi = e / jnp.sum(e)
    pi1 = jnp.sum(pi, axis=1).reshape(M, 1)
    pi2 = jnp.sum(pi, axis=0).reshape(M, 1)
    f1b = f1_ref[...]
    f2b = f2_ref[...]
    eu = jnp.sum(pi1 * f1b, axis=0, keepdims=True)
    ev = jnp.sum(pi2 * f2b, axis=0, keepdims=True)
    mm = jnp.dot(pi, f2b, preferred_element_type=jnp.float32)
    euv = jnp.sum(f1b * mm, axis=0, keepdims=True)
    fp = eu + ev - euv
    av = a_ref[...]
    res = av + fp - av * fp
    o_ref[...] = res[:, :N]

  a_pad = jnp.pad(a0, (0, NP - N)).reshape(1, NP)
  out = pl.pallas_call(
      body,
      out_shape=jax.ShapeDtypeStruct((1, N), jnp.float32),
  )(a_pad, w, f1, f2)
  return out.reshape(N)


def kernel(a0, W, X1, X2):
  x1 = jnp.transpose(X1, (0, 2, 3, 1))
  x2 = jnp.transpose(X2, (0, 2, 3, 1))
  pad = ((0, 0), (0, 0), (0, 0), (0, TAILP - (N - TAIL_N0)))
  x1t = jnp.pad(x1[:, :, :, TAIL_N0:], pad)
  x2t = jnp.pad(x2[:, :, :, TAIL_N0:], pad)
  f1, f2 = _clause_rows_sc(a0, x1, x2, x1t, x2t)
  return _combine_tc(a0, W, f1, f2)
